# Initial kernel scaffold; baseline (speedup 1.0000x reference)
import jax
import jax.numpy as jnp
from jax.experimental import pallas as pl


def kernel(x, w1, b1, w2, b2):
    raise NotImplementedError("write your pallas kernel here")


# speedup vs baseline: 1.1798x; 1.1798x over previous
import functools

import jax
import jax.numpy as jnp
from jax.experimental import pallas as pl
from jax.experimental.pallas import tpu as pltpu


def _ca_fused_kernel(x_ref, w1t_ref, b1_ref, w2_ref, b2_ref, o_ref, *, inv_hw):
    x = x_ref[0]
    pooled = jnp.sum(x, axis=1, keepdims=True,
                     dtype=jnp.float32) * inv_hw
    h = jnp.sum(w1t_ref[...] * pooled, axis=0, keepdims=True)
    h = jnp.maximum(h + b1_ref[...], 0.0)
    z = jnp.sum(w2_ref[...] * h, axis=1, keepdims=True)
    att = jax.nn.sigmoid(z + b2_ref[...])
    o_ref[0] = x * att.astype(x.dtype)


def kernel(x, w1, b1, w2, b2):
    N, C, H, W = x.shape
    Cr = w1.shape[0]
    HW = H * W

    x_flat = x.reshape(N, C, HW)
    w1t = jnp.transpose(w1)

    out_flat = pl.pallas_call(
        functools.partial(_ca_fused_kernel, inv_hw=float(1.0 / HW)),
        out_shape=jax.ShapeDtypeStruct((N, C, HW), x.dtype),
        grid=(N,),
        in_specs=[
            pl.BlockSpec((1, C, HW), lambda n: (n, 0, 0)),
            pl.BlockSpec((C, Cr), lambda n: (0, 0)),
            pl.BlockSpec((1, Cr), lambda n: (0, 0)),
            pl.BlockSpec((C, Cr), lambda n: (0, 0)),
            pl.BlockSpec((C, 1), lambda n: (0, 0)),
        ],
        out_specs=pl.BlockSpec((1, C, HW), lambda n: (n, 0, 0)),
        compiler_params=pltpu.CompilerParams(
            dimension_semantics=("parallel",),
            vmem_limit_bytes=64 << 20),
        cost_estimate=pl.CostEstimate(
            flops=2 * N * C * HW + 4 * N * C * Cr,
            transcendentals=N * C,
            bytes_accessed=2 * N * C * HW * 4 + (2 * C * Cr + Cr + C) * 4),
    )(x_flat, w1t, b1.reshape(1, Cr), w2, b2.reshape(C, 1))

    return out_flat.reshape(N, C, H, W)


# speedup vs baseline: 1.2551x; 1.0638x over previous
import functools

import jax
import jax.numpy as jnp
from jax.experimental import pallas as pl
from jax.experimental.pallas import tpu as pltpu


def _ca_fused_kernel(x_ref, w1t_ref, b1_ref, w2_ref, b2_ref, o_ref, *, inv_hw,
                     nb):
    for i in range(nb):
        x = x_ref[i]
        pooled = jnp.sum(x, axis=1, keepdims=True,
                         dtype=jnp.float32) * inv_hw
        h = jnp.sum(w1t_ref[...] * pooled, axis=0,
                    keepdims=True)
        h = jnp.maximum(h + b1_ref[...], 0.0)
        z = jnp.sum(w2_ref[...] * h, axis=1, keepdims=True)
        att = jax.nn.sigmoid(z + b2_ref[...])
        o_ref[i] = x * att.astype(x.dtype)


def kernel(x, w1, b1, w2, b2):
    N, C, H, W = x.shape
    Cr = w1.shape[0]
    HW = H * W

    x_flat = x.reshape(N, C, HW)
    w1t = jnp.transpose(w1)

    nb = 1
    for cand in (4, 2):
        if N % cand == 0 and cand * C * HW * 4 <= (10 << 20):
            nb = cand
            break

    out_flat = pl.pallas_call(
        functools.partial(_ca_fused_kernel, inv_hw=float(1.0 / HW), nb=nb),
        out_shape=jax.ShapeDtypeStruct((N, C, HW), x.dtype),
        grid=(N // nb,),
        in_specs=[
            pl.BlockSpec((nb, C, HW), lambda n: (n, 0, 0)),
            pl.BlockSpec((C, Cr), lambda n: (0, 0)),
            pl.BlockSpec((1, Cr), lambda n: (0, 0)),
            pl.BlockSpec((C, Cr), lambda n: (0, 0)),
            pl.BlockSpec((C, 1), lambda n: (0, 0)),
        ],
        out_specs=pl.BlockSpec((nb, C, HW), lambda n: (n, 0, 0)),
        compiler_params=pltpu.CompilerParams(
            dimension_semantics=("parallel",),
            vmem_limit_bytes=64 << 20),
        cost_estimate=pl.CostEstimate(
            flops=2 * N * C * HW + 4 * N * C * Cr,
            transcendentals=N * C,
            bytes_accessed=2 * N * C * HW * 4 + (2 * C * Cr + Cr + C) * 4),
    )(x_flat, w1t, b1.reshape(1, Cr), w2, b2.reshape(C, 1))

    return out_flat.reshape(N, C, H, W)


# speedup vs baseline: 5.6947x; 4.5371x over previous
import functools

import jax
import jax.numpy as jnp
from jax.experimental import pallas as pl
from jax.experimental.pallas import tpu as pltpu


def _ca_kernel(x_ref, w1t_ref, b1_ref, w2t_ref, b2_ref, o_ref, *, inv_hw):
    xb = x_ref[...]
    pooled = jnp.sum(xb, axis=0, dtype=jnp.float32) * inv_hw
    h = jnp.dot(pooled, w1t_ref[...],
                preferred_element_type=jnp.float32)
    h = jnp.maximum(h + b1_ref[...], 0.0)
    z = jnp.dot(h, w2t_ref[...],
                preferred_element_type=jnp.float32)
    att = jax.nn.sigmoid(z + b2_ref[...])
    o_ref[...] = xb * att[None].astype(xb.dtype)


def kernel(x, w1, b1, w2, b2):
    N, C, H, W = x.shape
    Cr = w1.shape[0]
    HW = H * W

    x_t = jnp.transpose(x, (2, 3, 0, 1)).reshape(HW, N, C)

    w1t = jnp.transpose(w1)
    w2t = jnp.transpose(w2)

    nb = 8 if N % 8 == 0 else N
    while 4 * HW * nb * C * 4 > (52 << 20) and nb % 2 == 0:
        nb //= 2

    out_t = pl.pallas_call(
        functools.partial(_ca_kernel, inv_hw=float(1.0 / HW)),
        out_shape=jax.ShapeDtypeStruct((HW, N, C), x.dtype),
        grid=(N // nb,),
        in_specs=[
            pl.BlockSpec((HW, nb, C), lambda n: (0, n, 0)),
            pl.BlockSpec((C, Cr), lambda n: (0, 0)),
            pl.BlockSpec((1, Cr), lambda n: (0, 0)),
            pl.BlockSpec((Cr, C), lambda n: (0, 0)),
            pl.BlockSpec((1, C), lambda n: (0, 0)),
        ],
        out_specs=pl.BlockSpec((HW, nb, C), lambda n: (0, n, 0)),
        compiler_params=pltpu.CompilerParams(
            dimension_semantics=("arbitrary",),
            vmem_limit_bytes=58 << 20),
        cost_estimate=pl.CostEstimate(
            flops=2 * N * C * HW + 4 * N * C * Cr,
            transcendentals=N * C,
            bytes_accessed=2 * N * C * HW * 4 + (2 * C * Cr + Cr + C) * 4),
    )(x_t, w1t, b1.reshape(1, Cr), w2t, b2.reshape(1, C))

    return out_t.reshape(H, W, N, C).transpose(2, 3, 0, 1)


# speedup vs baseline: 5.7014x; 1.0012x over previous
import functools

import jax
import jax.numpy as jnp
from jax.experimental import pallas as pl
from jax.experimental.pallas import tpu as pltpu


def _ca_kernel(x_ref, w1_ref, b1_ref, w2_ref, b2_ref, o_ref, *, inv_hw):
    xb = x_ref[...]
    pooled = jnp.sum(xb, axis=0, dtype=jnp.float32) * inv_hw
    h = jax.lax.dot_general(pooled, w1_ref[...],
                            (((1,), (1,)), ((), ())),
                            preferred_element_type=jnp.float32)
    h = jnp.maximum(h + b1_ref[...], 0.0)
    z = jax.lax.dot_general(h, w2_ref[...],
                            (((1,), (1,)), ((), ())),
                            preferred_element_type=jnp.float32)
    att = jax.nn.sigmoid(z + b2_ref[...])
    o_ref[...] = xb * att[None].astype(xb.dtype)


def kernel(x, w1, b1, w2, b2):
    N, C, H, W = x.shape
    Cr = w1.shape[0]
    HW = H * W

    x_t = jnp.transpose(x, (2, 3, 0, 1)).reshape(HW, N, C)

    nb = 8 if N % 8 == 0 else N
    while 4 * HW * nb * C * 4 > (52 << 20) and nb % 2 == 0:
        nb //= 2

    out_t = pl.pallas_call(
        functools.partial(_ca_kernel, inv_hw=float(1.0 / HW)),
        out_shape=jax.ShapeDtypeStruct((HW, N, C), x.dtype),
        grid=(N // nb,),
        in_specs=[
            pl.BlockSpec((HW, nb, C), lambda n: (0, n, 0)),
            pl.BlockSpec((Cr, C), lambda n: (0, 0)),
            pl.BlockSpec((1, Cr), lambda n: (0, 0)),
            pl.BlockSpec((C, Cr), lambda n: (0, 0)),
            pl.BlockSpec((1, C), lambda n: (0, 0)),
        ],
        out_specs=pl.BlockSpec((HW, nb, C), lambda n: (0, n, 0)),
        compiler_params=pltpu.CompilerParams(
            dimension_semantics=("arbitrary",),
            vmem_limit_bytes=58 << 20),
        cost_estimate=pl.CostEstimate(
            flops=2 * N * C * HW + 4 * N * C * Cr,
            transcendentals=N * C,
            bytes_accessed=2 * N * C * HW * 4 + (2 * C * Cr + Cr + C) * 4),
    )(x_t, w1, b1.reshape(1, Cr), w2, b2.reshape(1, C))

    return out_t.reshape(H, W, N, C).transpose(2, 3, 0, 1)
